# Initial kernel scaffold; baseline (speedup 1.0000x reference)
import jax
import jax.numpy as jnp
from jax.experimental import pallas as pl


def kernel(x, edge_index, edge_attr, We, be, W1, b1, W2, b2, gamma, beta):
    raise NotImplementedError("write your pallas kernel here")


# speedup vs baseline: 2.5110x; 2.5110x over previous
import functools

import jax
import jax.numpy as jnp
from jax import lax
from jax.experimental import pallas as pl
from jax.experimental.pallas import tpu as pltpu
from jax.experimental.pallas import tpu_sc as plsc

NC = 2
NS = 16
NW = NC * NS


def _edge_proj_body(ea_ref, we_ref, be_ref, e_ref):
    e_ref[...] = (
        jnp.dot(ea_ref[...], we_ref[...], preferred_element_type=jnp.float32)
        + be_ref[...]
    )


@functools.partial(jax.jit, static_argnames=("block",))
def _edge_proj(edge_attr, we, be, block=8000):
    E, ED = edge_attr.shape
    D = we.shape[1]
    grid = E // block
    return pl.pallas_call(
        _edge_proj_body,
        grid=(grid,),
        in_specs=[
            pl.BlockSpec((block, ED), lambda i: (i, 0)),
            pl.BlockSpec((ED, D), lambda i: (0, 0)),
            pl.BlockSpec((1, D), lambda i: (0, 0)),
        ],
        out_specs=pl.BlockSpec((block, D), lambda i: (i, 0)),
        out_shape=jax.ShapeDtypeStruct((E, D), jnp.float32),
    )(edge_attr, we, be)


def _node_body(h_ref, p_ref, w1_ref, b1_ref, w2_ref, b2_ref, g_ref, bt_ref,
               o_ref):
    N = h_ref.shape[0]
    z = h_ref[...] + p_ref[0, :N] + p_ref[1, :N]
    z = jnp.maximum(
        jnp.dot(z, w1_ref[...], preferred_element_type=jnp.float32)
        + b1_ref[...], 0.0)
    z = (jnp.dot(z, w2_ref[...], preferred_element_type=jnp.float32)
         + b2_ref[...])
    mu = jnp.mean(z, axis=0, keepdims=True)
    zc = z - mu
    var = jnp.mean(zc * zc, axis=0, keepdims=True)
    zn = zc * lax.rsqrt(var + 1e-5)
    o_ref[...] = jnp.maximum(zn * g_ref[...] + bt_ref[...], 0.0)


@jax.jit
def _node_update(h, parts, w1, b1, w2, b2, g, bt):
    N, D = h.shape
    return pl.pallas_call(
        _node_body,
        out_shape=jax.ShapeDtypeStruct((N, D), jnp.float32),
    )(h, parts, w1, b1, w2, b2, g, bt)


def _make_sc_agg(N, D, E, C):
    EPW = E // NW
    NCHUNK = EPW // C
    RPT = -(-N // (NS * 8)) * 8
    NPAD = RPT * NS
    mesh = plsc.VectorSubcoreMesh(core_axis_name="c", subcore_axis_name="s",
                                  num_cores=NC)

    def body(h_hbm, src_hbm, dst_hbm, e_hbm, parts_hbm,
             agg_sh, idx_s, idx_d, hbuf, ebuf, sem):
        cid = lax.axis_index("c")
        sid = lax.axis_index("s")
        wid = sid * NC + cid

        @pl.loop(0, C)
        def _zero_rows(r):
            for c8 in range(D // 16):
                ebuf[r, pl.ds(c8 * 16, 16)] = jnp.zeros((16,), jnp.float32)

        row0 = pl.multiple_of(sid * RPT, 8)
        full = RPT // C
        rem = RPT - full * C
        for k in range(full):
            pltpu.sync_copy(ebuf.at[pl.ds(0, C)],
                            agg_sh.at[pl.ds(row0 + k * C, C)])
        if rem:
            pltpu.sync_copy(ebuf.at[pl.ds(0, rem)],
                            agg_sh.at[pl.ds(row0 + full * C, rem)])
        plsc.subcore_barrier()

        base = wid * EPW

        @pl.loop(0, NCHUNK)
        def _chunk(j):
            off = pl.multiple_of(base + j * C, 8)
            pltpu.sync_copy(src_hbm.at[pl.ds(off, C)], idx_s)
            pltpu.sync_copy(dst_hbm.at[pl.ds(off, C)], idx_d)
            pltpu.async_copy(h_hbm.at[idx_s], hbuf, sem).wait()
            pltpu.sync_copy(e_hbm.at[pl.ds(off, C)], ebuf)

            @pl.loop(0, C)
            def _rows(r):
                for c8 in range(D // 16):
                    sl = pl.ds(c8 * 16, 16)
                    hbuf[r, sl] = jnp.maximum(hbuf[r, sl] + ebuf[r, sl], 0.0)

            pltpu.sync_copy(hbuf, agg_sh.at[idx_d], add=True)

        plsc.subcore_barrier()

        pltpu.sync_copy(agg_sh.at[pl.ds(row0, RPT)],
                        parts_hbm.at[cid, pl.ds(row0, RPT)])

    return pl.kernel(
        body,
        out_type=jax.ShapeDtypeStruct((NC, NPAD, D), jnp.float32),
        mesh=mesh,
        scratch_types=[
            pltpu.VMEM_SHARED((NPAD, D), jnp.float32),
            pltpu.VMEM((C,), jnp.int32),
            pltpu.VMEM((C,), jnp.int32),
            pltpu.VMEM((C, D), jnp.float32),
            pltpu.VMEM((C, D), jnp.float32),
            pltpu.SemaphoreType.DMA,
        ],
    )


def kernel(x, edge_index, edge_attr, We, be, W1, b1, W2, b2, gamma, beta):
    N, D = x.shape
    E = edge_attr.shape[0]
    L = We.shape[0]
    src = edge_index[0]
    dst = edge_index[1]
    sc_agg = _make_sc_agg(N, D, E, C=80)

    h = x
    for l in range(L):
        e = _edge_proj(edge_attr, We[l], be[l].reshape(1, -1))
        parts = sc_agg(h, src, dst, e)
        h = _node_update(h, parts, W1[l], b1[l].reshape(1, -1),
                         W2[l], b2[l].reshape(1, -1),
                         gamma[l].reshape(1, -1), beta[l].reshape(1, -1))
    return h
